# Initial kernel scaffold; baseline (speedup 1.0000x reference)
import jax
import jax.numpy as jnp
from jax.experimental import pallas as pl


def kernel(type, lanes_mid, crosswalks, lanes, emb_table):
    raise NotImplementedError("write your pallas kernel here")


# speedup vs baseline: 5.4331x; 5.4331x over previous
import jax
import jax.numpy as jnp
from jax import lax
from jax.experimental import pallas as pl
from jax.experimental.pallas import tpu as pltpu
from jax.experimental.pallas import tpu_sc as plsc

_TL_UNKNOWN = 5
_CROSSWALK = 10
_LANE_BDRY_LEFT = 11
_LANE_BDRY_RIGHT = 12
_NUM_TYPES = 13
_D = 64

_B = 4096
_TOTAL = 194
_BDRY_START = 66

_NC, _NS, _L = 2, 16, 16
_NW = _NC * _NS
_RPT = _B // _NW
_KB = 2
_NGRP = _RPT // _KB


def _body(tl_hbm, tpad_hbm, table_hbm, out_hbm,
          tl_v, idx_v, rows_v, buf0_v, buf1_v, table_v,
          gsem, sem0, sem1):
    wid = lax.axis_index("s") * _NC + lax.axis_index("c")
    b0 = wid * _RPT

    pltpu.sync_copy(tl_hbm.at[pl.ds(b0, _RPT)], tl_v)
    pltpu.sync_copy(table_hbm, table_v)

    for q in range(_RPT // _L):
        t16 = tl_v[pl.ds(q * _L, _L)]
        i16 = jnp.clip(t16.astype(jnp.int32) + _TL_UNKNOWN, 0, _NUM_TYPES - 1)
        idx_v[pl.ds(q * _L, _L)] = i16

    rows_cp = pltpu.async_copy(tpad_hbm.at[idx_v], rows_v, gsem)

    r10 = [table_v[_CROSSWALK, pl.ds(q * _L, _L)] for q in range(_D // _L)]
    r11 = [table_v[_LANE_BDRY_LEFT, pl.ds(q * _L, _L)] for q in range(_D // _L)]
    r12 = [table_v[_LANE_BDRY_RIGHT, pl.ds(q * _L, _L)] for q in range(_D // _L)]

    def fill(c, carry):
        is_cw = c < _BDRY_START
        is_right = ((c - _BDRY_START) & 1) == 1
        for q in range(_D // _L):
            v = jnp.where(is_cw, r10[q], jnp.where(is_right, r12[q], r11[q]))
            for k in range(_KB):
                buf0_v[k, c, pl.ds(q * _L, _L)] = v
                buf1_v[k, c, pl.ds(q * _L, _L)] = v
        return carry

    lax.fori_loop(2, _TOTAL, fill, 0)
    rows_cp.wait()

    bufs = (buf0_v, buf1_v)
    sems = (sem0, sem1)
    out_cps = [None, None]
    for g in range(_NGRP):
        buf = bufs[g & 1]
        if out_cps[g & 1] is not None:
            out_cps[g & 1].wait()
        for j in range(_KB):
            r = g * _KB + j
            for q in range(_D // _L):
                v = rows_v[r, pl.ds(q * _L, _L)]
                buf[j, 0, pl.ds(q * _L, _L)] = v
                buf[j, 1, pl.ds(q * _L, _L)] = v
        out_cps[g & 1] = pltpu.async_copy(
            buf, out_hbm.at[pl.ds(b0 + g * _KB, _KB), :, :], sems[g & 1])
    out_cps[0].wait()
    out_cps[1].wait()


@jax.jit
def _emb_fill(tl, table_pad, table):
    fn = pl.kernel(
        _body,
        out_type=jax.ShapeDtypeStruct((_B, _TOTAL, _D), jnp.float32),
        mesh=plsc.VectorSubcoreMesh(core_axis_name="c", subcore_axis_name="s"),
        scratch_types=[
            pltpu.VMEM((_RPT,), jnp.float32),
            pltpu.VMEM((_RPT,), jnp.int32),
            pltpu.VMEM((_RPT, 2 * _D), jnp.float32),
            pltpu.VMEM((_KB, _TOTAL, _D), jnp.float32),
            pltpu.VMEM((_KB, _TOTAL, _D), jnp.float32),
            pltpu.VMEM((_NUM_TYPES, _D), jnp.float32),
            pltpu.SemaphoreType.DMA,
            pltpu.SemaphoreType.DMA,
            pltpu.SemaphoreType.DMA,
        ],
    )
    return fn(tl, table_pad, table)


def kernel(type, lanes_mid, crosswalks, lanes, emb_table):
    del type, crosswalks, lanes
    tl = lanes_mid[:, 0, 0, -1]
    table_pad = jnp.pad(emb_table, ((0, 0), (0, _D)))
    return _emb_fill(tl, table_pad, emb_table)
